# baseline (speedup 1.0000x reference)
import functools

import jax
import jax.numpy as jnp
from jax import lax
from jax.experimental import pallas as pl
from jax.experimental.pallas import tpu as pltpu
from jax.experimental.pallas import tpu_sc as plsc

_EPS = 1e-6
_NC, _NS = 2, 16
_NW = _NC * _NS
_CHUNK = 128
_L = 16


def _sc_gather_t(users, pos, neg2, utf, uf, vf, B, D, NNEG, NUSER, NJOB):
    rows_per_w = B // _NW
    n_chunks = rows_per_w // _CHUNK
    JROWS = NNEG * D

    mesh = plsc.VectorSubcoreMesh(core_axis_name="c", subcore_axis_name="s")

    @functools.partial(
        pl.kernel,
        out_type=(
            jax.ShapeDtypeStruct((D, B), jnp.float32),
            jax.ShapeDtypeStruct((D, B), jnp.float32),
            jax.ShapeDtypeStruct((JROWS, B), jnp.float32),
        ),
        mesh=mesh,
        scratch_types=(
            pltpu.VMEM((_CHUNK,), jnp.int32),
            pltpu.VMEM((_CHUNK,), jnp.int32),
            pltpu.VMEM((NNEG, _CHUNK), jnp.int32),
            pltpu.VMEM((D, _CHUNK), jnp.int32),
            pltpu.VMEM((2 * D, _CHUNK), jnp.int32),
            pltpu.VMEM((D, _CHUNK), jnp.int32),
            pltpu.VMEM((JROWS, _CHUNK), jnp.int32),
            pltpu.VMEM((D, _CHUNK), jnp.float32),
            pltpu.VMEM((2 * D, _CHUNK), jnp.float32),
            pltpu.VMEM((D, _CHUNK), jnp.float32),
            pltpu.VMEM((D, _CHUNK), jnp.float32),
            pltpu.VMEM((JROWS, _CHUNK), jnp.float32),
            pltpu.SemaphoreType.DMA,
        ),
    )
    def k(users_h, pos_h, neg_h, utf_h, uf_h, vf_h,
          up_out, i_out, j_out,
          uv, pv, nv, ixu, ixuu, ixp, ixj, tut, tuu, tup, tpos, tneg, sem):
        wid = lax.axis_index("s") * _NC + lax.axis_index("c")
        for ci in range(n_chunks):
            base = wid * rows_per_w + ci * _CHUNK
            pltpu.sync_copy(users_h.at[pl.ds(base, _CHUNK)], uv)
            pltpu.sync_copy(pos_h.at[pl.ds(base, _CHUNK)], pv)
            pltpu.sync_copy(neg_h.at[:, pl.ds(base, _CHUNK)], nv)

            @pl.loop(0, D)
            def _(d):
                offu = d * NUSER
                offv = d * NJOB
                for g in range(_CHUNK // _L):
                    sl = pl.ds(g * _L, _L)
                    idsu = uv[sl]
                    ixu[d, sl] = idsu + offu
                    ixuu[d, sl] = idsu + offu
                    ixuu[D + d, sl] = idsu + (offu + D * NUSER)
                    ixp[d, sl] = pv[sl] + offv

            @pl.loop(0, JROWS)
            def _(m):
                kn = m // D
                offv = (m - kn * D) * NJOB
                for g in range(_CHUNK // _L):
                    sl = pl.ds(g * _L, _L)
                    ixj[m, sl] = nv[kn, sl] + offv

            @pl.loop(0, D)
            def _(d):
                pltpu.async_copy(utf_h.at[ixu.at[d]], tut.at[d], sem)
                pltpu.async_copy(vf_h.at[ixp.at[d]], tpos.at[d], sem)

            @pl.loop(0, 2 * D)
            def _(m):
                pltpu.async_copy(uf_h.at[ixuu.at[m]], tuu.at[m], sem)

            @pl.loop(0, JROWS)
            def _(m):
                pltpu.async_copy(vf_h.at[ixj.at[m]], tneg.at[m], sem)

            @pl.loop(0, D)
            def _(d):
                pltpu.make_async_copy(utf_h.at[ixu.at[d]], tut.at[d], sem).wait()

            @pl.loop(0, 2 * D)
            def _(m):
                pltpu.make_async_copy(uf_h.at[ixuu.at[m]], tuu.at[m], sem).wait()

            @pl.loop(0, D)
            def _(d):
                for g in range(_CHUNK // _L):
                    sl = pl.ds(g * _L, _L)
                    tup[d, sl] = (tut[d, sl] * 2.0 + tuu[d, sl]
                                  + tuu[D + d, sl])

            @pl.loop(0, D)
            def _(d):
                pltpu.make_async_copy(vf_h.at[ixp.at[d]], tpos.at[d], sem).wait()

            @pl.loop(0, JROWS)
            def _(m):
                pltpu.make_async_copy(vf_h.at[ixj.at[m]], tneg.at[m], sem).wait()

            pltpu.sync_copy(tup, up_out.at[:, pl.ds(base, _CHUNK)])
            pltpu.sync_copy(tpos, i_out.at[:, pl.ds(base, _CHUNK)])
            pltpu.sync_copy(tneg, j_out.at[:, pl.ds(base, _CHUNK)])

    return k(users, pos, neg2, utf, uf, vf)


def _tc_loss_t(upt, it, jt, das2, w, dab_t):
    D, B = upt.shape
    JROWS = jt.shape[0]
    NNEG = JROWS // D
    DAP = dab_t.shape[1]
    BLK = 512
    grid = B // BLK

    def body(up_ref, i_ref, j_ref, das_ref, w_ref, dab_ref, out_ref):
        onehot = (
            lax.broadcasted_iota(jnp.int32, (DAP, BLK), 0) == das_ref[...]
        ).astype(jnp.float32)
        u = jnp.dot(w_ref[...], up_ref[...],
                    preferred_element_type=jnp.float32)
        u = u + jnp.dot(dab_ref[...], onehot,
                        preferred_element_type=jnp.float32)
        dpos = u - i_ref[...] + _EPS
        dp = jnp.sqrt(jnp.sum(dpos * dpos, axis=0))
        acc = jnp.zeros((), jnp.float32)
        for kn in range(NNEG):
            dneg = u - j_ref[kn * D:(kn + 1) * D, :] + _EPS
            dn = jnp.sqrt(jnp.sum(dneg * dneg, axis=0))
            acc = acc + jnp.sum(jnp.maximum(dp - dn + 1.0, 0.0))

        @pl.when(pl.program_id(0) == 0)
        def _():
            out_ref[...] = jnp.zeros_like(out_ref)

        out_ref[...] += (acc * (1.0 / B)).reshape(1, 1)

    out = pl.pallas_call(
        body,
        grid=(grid,),
        in_specs=[
            pl.BlockSpec((D, BLK), lambda i: (0, i)),
            pl.BlockSpec((D, BLK), lambda i: (0, i)),
            pl.BlockSpec((JROWS, BLK), lambda i: (0, i)),
            pl.BlockSpec((1, BLK), lambda i: (0, i)),
            pl.BlockSpec((D, D), lambda i: (0, 0)),
            pl.BlockSpec((D, DAP), lambda i: (0, 0)),
        ],
        out_specs=pl.BlockSpec((1, 1), lambda i: (0, 0)),
        out_shape=jax.ShapeDtypeStruct((1, 1), jnp.float32),
    )(upt, it, jt, das2, w, dab_t)
    return out[0, 0]


def kernel(phase, users, pos_job_ids, behavior_ids, das, neg_job_id_lists,
           U_true, U, V, da_tab, W, b):
    del phase, behavior_ids
    NUSER, BEHm1, D = U.shape
    NJOB = V.shape[0]
    B = users.shape[0]
    NNEG = neg_job_id_lists.shape[1]
    DA = da_tab.shape[0] - 1
    utf = U_true.T.reshape(-1)
    uf = jnp.transpose(U, (1, 2, 0)).reshape(-1)
    vf = V.T.reshape(-1)
    neg2 = neg_job_id_lists.T
    das_c = jnp.clip(das, 0, DA).astype(jnp.int32).reshape(1, B)
    DAP = 128
    dab = jnp.zeros((DAP, D), jnp.float32).at[:DA + 1].set(da_tab + b[None, :])
    upt, it, jt = _sc_gather_t(users, pos_job_ids, neg2, utf, uf, vf,
                               B, D, NNEG, NUSER, NJOB)
    return _tc_loss_t(upt, it, jt, das_c, W, dab.T)


# speedup vs baseline: 6.9303x; 6.9303x over previous
import functools

import jax
import jax.numpy as jnp
from jax import lax
from jax.experimental import pallas as pl
from jax.experimental.pallas import tpu as pltpu
from jax.experimental.pallas import tpu_sc as plsc

_EPS = 1e-6
_NC, _NS = 2, 16
_NW = _NC * _NS
_CHUNK = 128
_L = 16


def _sc_gather(users, pos, negf, u_true_s, u3, v):
    B = users.shape[0]
    D = u_true_s.shape[1]
    NNEG = negf.shape[0] // B
    rows_per_w = B // _NW
    n_chunks = rows_per_w // _CHUNK

    mesh = plsc.VectorSubcoreMesh(core_axis_name="c", subcore_axis_name="s")

    @functools.partial(
        pl.kernel,
        out_type=(
            jax.ShapeDtypeStruct((B, D), jnp.float32),
            jax.ShapeDtypeStruct((B, D), jnp.float32),
            jax.ShapeDtypeStruct((NNEG * B, D), jnp.float32),
        ),
        mesh=mesh,
        scratch_types=(
            pltpu.VMEM((_CHUNK,), jnp.int32),
            pltpu.VMEM((_CHUNK,), jnp.int32),
            pltpu.VMEM((NNEG, _CHUNK), jnp.int32),
            pltpu.VMEM((_CHUNK, D), jnp.float32),
            pltpu.VMEM((_CHUNK, 2, D), jnp.float32),
            pltpu.VMEM((_CHUNK, D), jnp.float32),
            pltpu.VMEM((_CHUNK, D), jnp.float32),
            pltpu.VMEM((NNEG, _CHUNK, D), jnp.float32),
            pltpu.SemaphoreType.DMA,
        ),
        compiler_params=pltpu.CompilerParams(use_tc_tiling_on_sc=False),
    )
    def k(users_h, pos_h, neg_h, ut_tab, u3_tab, v_tab,
          up_out, i_out, j_out,
          uix, pix, nix, ut_b, uu_b, up_b, pi_b, nj_b, sem):
        wid = lax.axis_index("s") * _NC + lax.axis_index("c")
        for ci in range(n_chunks):
            base = wid * rows_per_w + ci * _CHUNK
            pltpu.sync_copy(users_h.at[pl.ds(base, _CHUNK)], uix)
            pltpu.sync_copy(pos_h.at[pl.ds(base, _CHUNK)], pix)
            for kn in range(NNEG):
                pltpu.sync_copy(neg_h.at[pl.ds(kn * B + base, _CHUNK)],
                                nix.at[kn])
            cps = [
                pltpu.async_copy(ut_tab.at[uix], ut_b, sem),
                pltpu.async_copy(u3_tab.at[uix], uu_b, sem),
                pltpu.async_copy(v_tab.at[pix], pi_b, sem),
            ]
            cps += [
                pltpu.async_copy(v_tab.at[nix.at[kn]], nj_b.at[kn], sem)
                for kn in range(NNEG)
            ]
            cps[0].wait()
            cps[1].wait()

            @pl.loop(0, _CHUNK)
            def _(r):
                for h in range(D // _L):
                    sl = pl.ds(h * _L, _L)
                    up_b[r, sl] = (ut_b[r, sl] * 2.0 + uu_b[r, 0, sl]
                                   + uu_b[r, 1, sl])

            for cp in cps[2:]:
                cp.wait()
            pltpu.sync_copy(up_b, up_out.at[pl.ds(base, _CHUNK)])
            pltpu.sync_copy(pi_b, i_out.at[pl.ds(base, _CHUNK)])
            for kn in range(NNEG):
                pltpu.sync_copy(nj_b.at[kn],
                                j_out.at[pl.ds(kn * B + base, _CHUNK)])

    return k(users, pos, negf, u_true_s, u3, v)


def _tc_loss(up, ig, jg, das2, wt, dab):
    B, D = up.shape
    NNEG = jg.shape[0] // B
    DAP = dab.shape[0]
    BLK = 1024
    grid = B // BLK

    def body(up_ref, i_ref, j_ref, das_ref, w_ref, dab_ref, out_ref):
        onehot = (
            lax.broadcasted_iota(jnp.int32, (BLK, DAP), 1) == das_ref[...]
        ).astype(jnp.float32)
        u = jnp.dot(up_ref[...], w_ref[...],
                    preferred_element_type=jnp.float32)
        u = u + jnp.dot(onehot, dab_ref[...],
                        preferred_element_type=jnp.float32)
        dpos = u - i_ref[...] + _EPS
        dp = jnp.sqrt(jnp.sum(dpos * dpos, axis=1))
        acc = jnp.zeros((), jnp.float32)
        for kn in range(NNEG):
            dneg = u - j_ref[kn] + _EPS
            dn = jnp.sqrt(jnp.sum(dneg * dneg, axis=1))
            acc = acc + jnp.sum(jnp.maximum(dp - dn + 1.0, 0.0))

        @pl.when(pl.program_id(0) == 0)
        def _():
            out_ref[...] = jnp.zeros_like(out_ref)

        out_ref[...] += (acc * (1.0 / B)).reshape(1, 1)

    out = pl.pallas_call(
        body,
        grid=(grid,),
        in_specs=[
            pl.BlockSpec((BLK, D), lambda i: (i, 0)),
            pl.BlockSpec((BLK, D), lambda i: (i, 0)),
            pl.BlockSpec((NNEG, BLK, D), lambda i: (0, i, 0)),
            pl.BlockSpec((BLK, 1), lambda i: (i, 0)),
            pl.BlockSpec((D, D), lambda i: (0, 0)),
            pl.BlockSpec((DAP, D), lambda i: (0, 0)),
        ],
        out_specs=pl.BlockSpec((1, 1), lambda i: (0, 0)),
        out_shape=jax.ShapeDtypeStruct((1, 1), jnp.float32),
    )(up, ig, jg.reshape(NNEG, B, D), das2, wt, dab)
    return out[0, 0]


def kernel(phase, users, pos_job_ids, behavior_ids, das, neg_job_id_lists,
           U_true, U, V, da_tab, W, b):
    del phase, behavior_ids
    NUSER, BEHm1, D = U.shape
    B = users.shape[0]
    DA = da_tab.shape[0] - 1
    u_true_s = U_true[:NUSER]
    negf = neg_job_id_lists.T.reshape(-1)
    das_c = jnp.clip(das, 0, DA).astype(jnp.int32).reshape(B, 1)
    DAP = 128
    dab = jnp.zeros((DAP, D), jnp.float32).at[:DA + 1].set(da_tab + b[None, :])
    up, ig, jg = _sc_gather(users, pos_job_ids, negf, u_true_s, U, V)
    return _tc_loss(up, ig, jg, das_c, W.T, dab)


# speedup vs baseline: 7.1754x; 1.0354x over previous
import functools

import jax
import jax.numpy as jnp
from jax import lax
from jax.experimental import pallas as pl
from jax.experimental.pallas import tpu as pltpu
from jax.experimental.pallas import tpu_sc as plsc

_EPS = 1e-6
_NC, _NS = 2, 16
_NW = _NC * _NS
_CHUNK = 128
_L = 16
_PK = 128


def _sc_gather(users, pos, negf, u_true_s, u3, v):
    B = users.shape[0]
    D = u_true_s.shape[1]
    NNEG = negf.shape[0] // B
    rows_per_w = B // _NW
    n_chunks = rows_per_w // _CHUNK

    mesh = plsc.VectorSubcoreMesh(core_axis_name="c", subcore_axis_name="s")

    @functools.partial(
        pl.kernel,
        out_type=(
            jax.ShapeDtypeStruct((B, _PK), jnp.float32),
            jax.ShapeDtypeStruct((B, _PK), jnp.float32),
        ),
        mesh=mesh,
        scratch_types=(
            pltpu.VMEM((_CHUNK,), jnp.int32),
            pltpu.VMEM((_CHUNK,), jnp.int32),
            pltpu.VMEM((NNEG, _CHUNK), jnp.int32),
            pltpu.VMEM((_CHUNK, D), jnp.float32),
            pltpu.VMEM((_CHUNK, 2, D), jnp.float32),
            pltpu.VMEM((_CHUNK, D), jnp.float32),
            pltpu.VMEM((_CHUNK, D), jnp.float32),
            pltpu.VMEM((NNEG, _CHUNK, D), jnp.float32),
            pltpu.SemaphoreType.DMA,
        ),
        compiler_params=pltpu.CompilerParams(use_tc_tiling_on_sc=False),
    )
    def k(users_h, pos_h, neg_h, ut_tab, u3_tab, v_tab,
          o1, o2, uix, pix, nix, ut_b, uu_b, up_b, pi_b, nj_b, sem):
        wid = lax.axis_index("s") * _NC + lax.axis_index("c")
        for ci in range(n_chunks):
            base = wid * rows_per_w + ci * _CHUNK
            rows = pl.ds(base, _CHUNK)
            pltpu.sync_copy(users_h.at[rows], uix)
            pltpu.sync_copy(pos_h.at[rows], pix)
            for kn in range(NNEG):
                pltpu.sync_copy(neg_h.at[pl.ds(kn * B + base, _CHUNK)],
                                nix.at[kn])
            cps = [
                pltpu.async_copy(ut_tab.at[uix], ut_b, sem),
                pltpu.async_copy(u3_tab.at[uix], uu_b, sem),
                pltpu.async_copy(v_tab.at[pix], pi_b, sem),
            ]
            cps += [
                pltpu.async_copy(v_tab.at[nix.at[kn]], nj_b.at[kn], sem)
                for kn in range(NNEG)
            ]
            cps[0].wait()
            cps[1].wait()

            @pl.loop(0, _CHUNK)
            def _(r):
                for h in range(D // _L):
                    sl = pl.ds(h * _L, _L)
                    up_b[r, sl] = (ut_b[r, sl] * 2.0 + uu_b[r, 0, sl]
                                   + uu_b[r, 1, sl])

            for cp in cps[2:]:
                cp.wait()
            pltpu.sync_copy(up_b, o1.at[rows, pl.ds(0, D)])
            pltpu.sync_copy(pi_b, o1.at[rows, pl.ds(D, D)])
            pltpu.sync_copy(nj_b.at[0], o1.at[rows, pl.ds(2 * D, D)])
            pltpu.sync_copy(nj_b.at[1], o1.at[rows, pl.ds(3 * D, D)])
            pltpu.sync_copy(nj_b.at[2], o2.at[rows, pl.ds(0, D)])
            pltpu.sync_copy(nj_b.at[3], o2.at[rows, pl.ds(D, D)])
            pltpu.sync_copy(nj_b.at[4], o2.at[rows, pl.ds(2 * D, D)])

    return k(users, pos, negf, u_true_s, u3, v)


def _tc_loss(p1, p2, das2, wt, dab):
    B = p1.shape[0]
    D = dab.shape[1]
    DAP = dab.shape[0]
    BLK = 1024
    grid = B // BLK

    def body(p1_ref, p2_ref, das_ref, w_ref, dab_ref, out_ref):
        x1 = p1_ref[...]
        x2 = p2_ref[...]
        onehot = (
            lax.broadcasted_iota(jnp.int32, (BLK, DAP), 1) == das_ref[...]
        ).astype(jnp.float32)
        u = jnp.dot(x1[:, 0:D], w_ref[...],
                    preferred_element_type=jnp.float32)
        u = u + jnp.dot(onehot, dab_ref[...],
                        preferred_element_type=jnp.float32)
        dpos = u - x1[:, D:2 * D] + _EPS
        dp = jnp.sqrt(jnp.sum(dpos * dpos, axis=1))
        negs = [x1[:, 2 * D:3 * D], x1[:, 3 * D:4 * D],
                x2[:, 0:D], x2[:, D:2 * D], x2[:, 2 * D:3 * D]]
        acc = jnp.zeros((), jnp.float32)
        for xj in negs:
            dneg = u - xj + _EPS
            dn = jnp.sqrt(jnp.sum(dneg * dneg, axis=1))
            acc = acc + jnp.sum(jnp.maximum(dp - dn + 1.0, 0.0))

        @pl.when(pl.program_id(0) == 0)
        def _():
            out_ref[...] = jnp.zeros_like(out_ref)

        out_ref[...] += (acc * (1.0 / B)).reshape(1, 1)

    out = pl.pallas_call(
        body,
        grid=(grid,),
        in_specs=[
            pl.BlockSpec((BLK, _PK), lambda i: (i, 0)),
            pl.BlockSpec((BLK, _PK), lambda i: (i, 0)),
            pl.BlockSpec((BLK, 1), lambda i: (i, 0)),
            pl.BlockSpec((D, D), lambda i: (0, 0)),
            pl.BlockSpec((DAP, D), lambda i: (0, 0)),
        ],
        out_specs=pl.BlockSpec((1, 1), lambda i: (0, 0)),
        out_shape=jax.ShapeDtypeStruct((1, 1), jnp.float32),
    )(p1, p2, das2, wt, dab)
    return out[0, 0]


def kernel(phase, users, pos_job_ids, behavior_ids, das, neg_job_id_lists,
           U_true, U, V, da_tab, W, b):
    del phase, behavior_ids
    NUSER, BEHm1, D = U.shape
    B = users.shape[0]
    DA = da_tab.shape[0] - 1
    u_true_s = U_true[:NUSER]
    negf = neg_job_id_lists.T.reshape(-1)
    das_c = jnp.clip(das, 0, DA).astype(jnp.int32).reshape(B, 1)
    DAP = 128
    dab = jnp.zeros((DAP, D), jnp.float32).at[:DA + 1].set(da_tab + b[None, :])
    p1, p2 = _sc_gather(users, pos_job_ids, negf, u_true_s, U, V)
    return _tc_loss(p1, p2, das_c, W.T, dab)
